# baseline (speedup 1.0000x reference)
import functools
from functools import partial

import jax
import jax.numpy as jnp
from jax.experimental import pallas as pl
from jax.experimental.pallas import tpu as pltpu

_CFG = [(256, 0.2, 32), (128, 0.4, 32), (64, 0.4, 32), (16, 0.8, 32)]
_HI = jax.lax.Precision.HIGHEST


def _fps(xyz, npoint):
    N = xyz.shape[0]

    def body(i, state):
        dists, farthest, idxs = state
        idxs = idxs.at[i].set(farthest)
        centroid = xyz[farthest]
        d = jnp.sum((xyz - centroid) ** 2, axis=-1)
        dists = jnp.minimum(dists, d)
        farthest = jnp.argmax(dists).astype(jnp.int32)
        return (dists, farthest, idxs)

    init = (jnp.full((N,), 1e10, dtype=xyz.dtype), jnp.array(0, jnp.int32),
            jnp.zeros((npoint,), jnp.int32))
    _, _, idxs = jax.lax.fori_loop(0, npoint, body, init)
    return idxs


def _fps_body(xt_ref, out_ref, dmin_ref, *, N, npoint):
    B = out_ref.shape[0]
    x = xt_ref[:, 0, :]
    y = xt_ref[:, 1, :]
    z = xt_ref[:, 2, :]
    iota_n = jax.lax.broadcasted_iota(jnp.int32, (B, N), 1)
    iota_p = jax.lax.broadcasted_iota(jnp.int32, (B, npoint), 1)
    dmin_ref[...] = jnp.full((B, N), 1e10, jnp.float32)
    out_ref[...] = jnp.zeros((B, npoint), jnp.int32)

    def body(i, far):
        out_ref[...] = jnp.where(iota_p == i, far, out_ref[...])
        oh = (iota_n == far).astype(jnp.float32)
        cx = jnp.sum(x * oh, axis=1, keepdims=True)
        cy = jnp.sum(y * oh, axis=1, keepdims=True)
        cz = jnp.sum(z * oh, axis=1, keepdims=True)
        dx = x - cx
        dy = y - cy
        dz = z - cz
        d = (dx * dx + dy * dy) + dz * dz
        dm = jnp.minimum(dmin_ref[...], d)
        dmin_ref[...] = dm
        mx = jnp.max(dm, axis=1, keepdims=True)
        nf = jnp.min(jnp.where(dm == mx, iota_n, N), axis=1, keepdims=True)
        return nf.astype(jnp.int32)

    jax.lax.fori_loop(0, npoint, body, jnp.zeros((B, 1), jnp.int32))


def _fps_pallas(xt, npoint):
    B, _, N = xt.shape
    return pl.pallas_call(
        partial(_fps_body, N=N, npoint=npoint),
        grid=(1,),
        in_specs=[pl.BlockSpec((B, 3, N), lambda g: (0, 0, 0))],
        out_specs=pl.BlockSpec((B, npoint), lambda g: (0, 0)),
        out_shape=jax.ShapeDtypeStruct((B, npoint), jnp.int32),
        scratch_shapes=[pltpu.VMEM((B, N), jnp.float32)],
    )(xt)


def _sa_body(pts_ref, xt_ref, nx_ref, w1x_ref, *wrefs, TM, K, N, r2):
    out_ref = wrefs[-1]
    wrefs = wrefs[:-1]
    pts = pts_ref[0]
    xt = xt_ref[0]
    nx = nx_ref[0]
    iota = jax.lax.broadcasted_iota(jnp.int32, (TM, N), 1)
    d = None
    for c in range(3):
        diff = nx[:, c:c + 1] - xt[c:c + 1, :]
        s = diff * diff
        d = s if d is None else d + s
    key = jnp.where(d < r2, iota, N)
    sels = []
    for _ in range(K):
        cur = jnp.min(key, axis=1, keepdims=True)
        key = jnp.where(key == cur, N + 1, key)
        sels.append(cur)
    first = sels[0]
    oh_parts = []
    for k in range(K):
        sel = first if k == 0 else jnp.where(sels[k] >= N, first, sels[k])
        oh_parts.append((iota == sel).astype(jnp.float32))
    oh = jnp.concatenate(oh_parts, axis=0)
    g = jax.lax.dot(oh, pts, precision=_HI)
    corr1 = jax.lax.dot(nx, w1x_ref[...], precision=_HI)
    corr = jnp.concatenate([corr1] * K, axis=0)
    h = jnp.maximum(jax.lax.dot(g, wrefs[0][...], precision=_HI) - corr, 0.0)
    for wr in wrefs[1:]:
        h = jnp.maximum(jax.lax.dot(h, wr[...], precision=_HI), 0.0)
    acc = h[0:TM]
    for k in range(1, K):
        acc = jnp.maximum(acc, h[k * TM:(k + 1) * TM])
    out_ref[0] = acc


def _sa_pallas(pts, xt, new_xyz, weights, radius, K):
    B, N, Cin = pts.shape
    M = new_xyz.shape[1]
    TM = min(64, M)
    NB = M // TM
    C1 = weights[0].shape[1]
    Cout = weights[-1].shape[1]
    w1x = weights[0][:3]
    in_specs = [
        pl.BlockSpec((1, N, Cin), lambda b, mb: (b, 0, 0)),
        pl.BlockSpec((1, 3, N), lambda b, mb: (b, 0, 0)),
        pl.BlockSpec((1, TM, 3), lambda b, mb: (b, mb, 0)),
        pl.BlockSpec((3, C1), lambda b, mb: (0, 0)),
    ] + [pl.BlockSpec(w.shape, lambda b, mb: (0, 0)) for w in weights]
    return pl.pallas_call(
        partial(_sa_body, TM=TM, K=K, N=N, r2=radius ** 2),
        grid=(B, NB),
        in_specs=in_specs,
        out_specs=pl.BlockSpec((1, TM, Cout), lambda b, mb: (b, mb, 0)),
        out_shape=jax.ShapeDtypeStruct((B, M, Cout), jnp.float32),
    )(pts, xt, new_xyz, w1x, *weights)


def _fp_body(u_ref, kt_ref, kf_ref, uf_ref, w1a_ref, w1b_ref, *wrefs,
             TN, m):
    out_ref = wrefs[-1]
    wrefs = wrefs[:-1]
    u = u_ref[0]
    kt = kt_ref[0]
    kf = kf_ref[0]
    uf = uf_ref[0]
    d = None
    for c in range(3):
        diff = u[:, c:c + 1] - kt[c:c + 1, :]
        s = diff * diff
        d = s if d is None else d + s
    iota = jax.lax.broadcasted_iota(jnp.int32, (TN, m), 1)
    picks, recips = [], []
    for _ in range(3):
        mn = jnp.min(d, axis=1, keepdims=True)
        ij = jnp.min(jnp.where(d == mn, iota, m), axis=1, keepdims=True)
        pick = iota == ij
        recips.append(1.0 / (mn + 1e-8))
        picks.append(pick)
        d = jnp.where(pick, 1e30, d)
    norm = recips[0] + recips[1] + recips[2]
    S = None
    for pick, r in zip(picks, recips):
        t = jnp.where(pick, r / norm, 0.0)
        S = t if S is None else S + t
    interp = jax.lax.dot(S, kf, precision=_HI)
    h = jax.lax.dot(interp, w1a_ref[...], precision=_HI) \
        + jax.lax.dot(uf, w1b_ref[...], precision=_HI)
    h = jnp.maximum(h, 0.0)
    for wr in wrefs:
        h = jnp.maximum(jax.lax.dot(h, wr[...], precision=_HI), 0.0)
    out_ref[0] = h


def _fp_pallas(unknown, known, uf, kf, weights):
    B, n, _ = unknown.shape
    m = known.shape[1]
    Ck = kf.shape[-1]
    Cu = uf.shape[-1]
    TN = min(512, n)
    NB = n // TN
    Cout = weights[-1].shape[1]
    kt = jnp.transpose(known, (0, 2, 1))
    w1a, w1b = weights[0][:Ck], weights[0][Ck:]
    rest = list(weights[1:])
    in_specs = [
        pl.BlockSpec((1, TN, 3), lambda b, nb: (b, nb, 0)),
        pl.BlockSpec((1, 3, m), lambda b, nb: (b, 0, 0)),
        pl.BlockSpec((1, m, Ck), lambda b, nb: (b, 0, 0)),
        pl.BlockSpec((1, TN, Cu), lambda b, nb: (b, nb, 0)),
        pl.BlockSpec(w1a.shape, lambda b, nb: (0, 0)),
        pl.BlockSpec(w1b.shape, lambda b, nb: (0, 0)),
    ] + [pl.BlockSpec(w.shape, lambda b, nb: (0, 0)) for w in rest]
    return pl.pallas_call(
        partial(_fp_body, TN=TN, m=m),
        grid=(B, NB),
        in_specs=in_specs,
        out_specs=pl.BlockSpec((1, TN, Cout), lambda b, nb: (b, nb, 0)),
        out_shape=jax.ShapeDtypeStruct((B, n, Cout), jnp.float32),
    )(unknown, kt, kf, uf, w1a, w1b, *rest)


def kernel(pointcloud, sa_params, fp_params):
    pc = pointcloud.reshape((-1,) + pointcloud.shape[-2:])
    xyz = pc[..., :3]
    l_xyz = [xyz]
    l_f = [pc[..., 3:]]
    for i, (npoint, radius, nsample) in enumerate(_CFG):
        xt = jnp.transpose(l_xyz[i], (0, 2, 1))
        fps_idx = _fps_pallas(xt, npoint)
        new_xyz = jnp.take_along_axis(l_xyz[i], fps_idx[:, :, None].astype(jnp.int32), axis=1)
        pts = jnp.concatenate([l_xyz[i], l_f[i]], axis=-1)
        l_xyz.append(new_xyz)
        l_f.append(_sa_pallas(pts, xt, new_xyz, sa_params[i], radius, nsample))
    for i in range(-1, -5, -1):
        l_f[i - 1] = _fp_pallas(l_xyz[i - 1], l_xyz[i], l_f[i - 1], l_f[i],
                                fp_params[i])
    return jnp.transpose(l_f[0], (0, 2, 1))


# speedup vs baseline: 1.2858x; 1.2858x over previous
import functools
from functools import partial

import jax
import jax.numpy as jnp
from jax.experimental import pallas as pl
from jax.experimental.pallas import tpu as pltpu

_CFG = [(256, 0.2, 32), (128, 0.4, 32), (64, 0.4, 32), (16, 0.8, 32)]
_HI = jax.lax.Precision.HIGHEST
_MLP = jax.lax.Precision.DEFAULT


def _fps(xyz, npoint):
    N = xyz.shape[0]

    def body(i, state):
        dists, farthest, idxs = state
        idxs = idxs.at[i].set(farthest)
        centroid = xyz[farthest]
        d = jnp.sum((xyz - centroid) ** 2, axis=-1)
        dists = jnp.minimum(dists, d)
        farthest = jnp.argmax(dists).astype(jnp.int32)
        return (dists, farthest, idxs)

    init = (jnp.full((N,), 1e10, dtype=xyz.dtype), jnp.array(0, jnp.int32),
            jnp.zeros((npoint,), jnp.int32))
    _, _, idxs = jax.lax.fori_loop(0, npoint, body, init)
    return idxs


def _fps_body(xt_ref, out_ref, dmin_ref, *, N, npoint):
    B = out_ref.shape[0]
    x = xt_ref[:, 0, :]
    y = xt_ref[:, 1, :]
    z = xt_ref[:, 2, :]
    iota_n = jax.lax.broadcasted_iota(jnp.int32, (B, N), 1)
    iota_p = jax.lax.broadcasted_iota(jnp.int32, (B, npoint), 1)
    dmin_ref[...] = jnp.full((B, N), 1e10, jnp.float32)
    out_ref[...] = jnp.zeros((B, npoint), jnp.int32)

    def body(i, far):
        out_ref[...] = jnp.where(iota_p == i, far, out_ref[...])
        oh = (iota_n == far).astype(jnp.float32)
        cx = jnp.sum(x * oh, axis=1, keepdims=True)
        cy = jnp.sum(y * oh, axis=1, keepdims=True)
        cz = jnp.sum(z * oh, axis=1, keepdims=True)
        dx = x - cx
        dy = y - cy
        dz = z - cz
        d = (dx * dx + dy * dy) + dz * dz
        dm = jnp.minimum(dmin_ref[...], d)
        dmin_ref[...] = dm
        mx = jnp.max(dm, axis=1, keepdims=True)
        nf = jnp.min(jnp.where(dm == mx, iota_n, N), axis=1, keepdims=True)
        return nf.astype(jnp.int32)

    jax.lax.fori_loop(0, npoint, body, jnp.zeros((B, 1), jnp.int32))


def _fps_pallas(xt, npoint):
    B, _, N = xt.shape
    return pl.pallas_call(
        partial(_fps_body, N=N, npoint=npoint),
        grid=(1,),
        in_specs=[pl.BlockSpec((B, 3, N), lambda g: (0, 0, 0))],
        out_specs=pl.BlockSpec((B, npoint), lambda g: (0, 0)),
        out_shape=jax.ShapeDtypeStruct((B, npoint), jnp.int32),
        scratch_shapes=[pltpu.VMEM((B, N), jnp.float32)],
    )(xt)


def _sa_body(pts_ref, xt_ref, nx_ref, w1x_ref, *wrefs, TM, K, N, r2):
    out_ref = wrefs[-1]
    wrefs = wrefs[:-1]
    pts = pts_ref[0]
    xt = xt_ref[0]
    nx = nx_ref[0]
    iota = jax.lax.broadcasted_iota(jnp.int32, (TM, N), 1)
    d = None
    for c in range(3):
        diff = nx[:, c:c + 1] - xt[c:c + 1, :]
        s = diff * diff
        d = s if d is None else d + s
    key = jnp.where(d < r2, iota, N)
    sels = []
    for _ in range(K):
        cur = jnp.min(key, axis=1, keepdims=True)
        key = jnp.where(key == cur, N + 1, key)
        sels.append(cur)
    first = sels[0]
    oh_parts = []
    for k in range(K):
        sel = first if k == 0 else jnp.where(sels[k] >= N, first, sels[k])
        oh_parts.append((iota == sel).astype(jnp.float32))
    oh = jnp.concatenate(oh_parts, axis=0)
    g = jax.lax.dot(oh, pts, precision=_HI)
    corr1 = jax.lax.dot(nx, w1x_ref[...], precision=_HI)
    corr = jnp.concatenate([corr1] * K, axis=0)
    h = jnp.maximum(jax.lax.dot(g, wrefs[0][...], precision=_MLP) - corr, 0.0)
    for wr in wrefs[1:]:
        h = jnp.maximum(jax.lax.dot(h, wr[...], precision=_MLP), 0.0)
    acc = h[0:TM]
    for k in range(1, K):
        acc = jnp.maximum(acc, h[k * TM:(k + 1) * TM])
    out_ref[0] = acc


def _sa_pallas(pts, xt, new_xyz, weights, radius, K):
    B, N, Cin = pts.shape
    M = new_xyz.shape[1]
    TM = min(32, M)
    NB = M // TM
    C1 = weights[0].shape[1]
    Cout = weights[-1].shape[1]
    w1x = weights[0][:3]
    in_specs = [
        pl.BlockSpec((1, N, Cin), lambda b, mb: (b, 0, 0)),
        pl.BlockSpec((1, 3, N), lambda b, mb: (b, 0, 0)),
        pl.BlockSpec((1, TM, 3), lambda b, mb: (b, mb, 0)),
        pl.BlockSpec((3, C1), lambda b, mb: (0, 0)),
    ] + [pl.BlockSpec(w.shape, lambda b, mb: (0, 0)) for w in weights]
    return pl.pallas_call(
        partial(_sa_body, TM=TM, K=K, N=N, r2=radius ** 2),
        grid=(B, NB),
        in_specs=in_specs,
        out_specs=pl.BlockSpec((1, TM, Cout), lambda b, mb: (b, mb, 0)),
        out_shape=jax.ShapeDtypeStruct((B, M, Cout), jnp.float32),
    )(pts, xt, new_xyz, w1x, *weights)


def _fp_body(u_ref, kt_ref, kf_ref, uf_ref, w1a_ref, w1b_ref, *wrefs,
             TN, m):
    out_ref = wrefs[-1]
    wrefs = wrefs[:-1]
    u = u_ref[0]
    kt = kt_ref[0]
    kf = kf_ref[0]
    uf = uf_ref[0]
    d = None
    for c in range(3):
        diff = u[:, c:c + 1] - kt[c:c + 1, :]
        s = diff * diff
        d = s if d is None else d + s
    iota = jax.lax.broadcasted_iota(jnp.int32, (TN, m), 1)
    picks, recips = [], []
    for _ in range(3):
        mn = jnp.min(d, axis=1, keepdims=True)
        ij = jnp.min(jnp.where(d == mn, iota, m), axis=1, keepdims=True)
        pick = iota == ij
        recips.append(1.0 / (mn + 1e-8))
        picks.append(pick)
        d = jnp.where(pick, 1e30, d)
    norm = recips[0] + recips[1] + recips[2]
    S = None
    for pick, r in zip(picks, recips):
        t = jnp.where(pick, r / norm, 0.0)
        S = t if S is None else S + t
    interp = jax.lax.dot(S, kf, precision=_HI)
    h = jax.lax.dot(interp, w1a_ref[...], precision=_MLP) \
        + jax.lax.dot(uf, w1b_ref[...], precision=_MLP)
    h = jnp.maximum(h, 0.0)
    for wr in wrefs:
        h = jnp.maximum(jax.lax.dot(h, wr[...], precision=_MLP), 0.0)
    out_ref[0] = h


def _fp_pallas(unknown, known, uf, kf, weights):
    B, n, _ = unknown.shape
    m = known.shape[1]
    Ck = kf.shape[-1]
    Cu = uf.shape[-1]
    TN = min(512, n)
    NB = n // TN
    Cout = weights[-1].shape[1]
    kt = jnp.transpose(known, (0, 2, 1))
    w1a, w1b = weights[0][:Ck], weights[0][Ck:]
    rest = list(weights[1:])
    in_specs = [
        pl.BlockSpec((1, TN, 3), lambda b, nb: (b, nb, 0)),
        pl.BlockSpec((1, 3, m), lambda b, nb: (b, 0, 0)),
        pl.BlockSpec((1, m, Ck), lambda b, nb: (b, 0, 0)),
        pl.BlockSpec((1, TN, Cu), lambda b, nb: (b, nb, 0)),
        pl.BlockSpec(w1a.shape, lambda b, nb: (0, 0)),
        pl.BlockSpec(w1b.shape, lambda b, nb: (0, 0)),
    ] + [pl.BlockSpec(w.shape, lambda b, nb: (0, 0)) for w in rest]
    return pl.pallas_call(
        partial(_fp_body, TN=TN, m=m),
        grid=(B, NB),
        in_specs=in_specs,
        out_specs=pl.BlockSpec((1, TN, Cout), lambda b, nb: (b, nb, 0)),
        out_shape=jax.ShapeDtypeStruct((B, n, Cout), jnp.float32),
    )(unknown, kt, kf, uf, w1a, w1b, *rest)


def kernel(pointcloud, sa_params, fp_params):
    pc = pointcloud.reshape((-1,) + pointcloud.shape[-2:])
    xyz = pc[..., :3]
    l_xyz = [xyz]
    l_f = [pc[..., 3:]]
    for i, (npoint, radius, nsample) in enumerate(_CFG):
        xt = jnp.transpose(l_xyz[i], (0, 2, 1))
        fps_idx = _fps_pallas(xt, npoint)
        new_xyz = jnp.take_along_axis(l_xyz[i], fps_idx[:, :, None].astype(jnp.int32), axis=1)
        pts = jnp.concatenate([l_xyz[i], l_f[i]], axis=-1)
        l_xyz.append(new_xyz)
        l_f.append(_sa_pallas(pts, xt, new_xyz, sa_params[i], radius, nsample))
    for i in range(-1, -5, -1):
        l_f[i - 1] = _fp_pallas(l_xyz[i - 1], l_xyz[i], l_f[i - 1], l_f[i],
                                fp_params[i])
    return jnp.transpose(l_f[0], (0, 2, 1))
